# baseline (speedup 1.0000x reference)
import functools

import jax
import jax.numpy as jnp
from jax import lax
from jax.experimental import pallas as pl
from jax.experimental.pallas import tpu as pltpu
from jax.experimental.pallas import tpu_sc as plsc

_SELU_ALPHA = 1.6732632423543772
_SELU_SCALE = 1.0507009873554805

_CH = 64
_SEG = 2000
_NG = 3
_BK = 6
_NQ = 2
_SB = 14


def _lane_bcast(v, i):
    dn = lax.GatherDimensionNumbers(
        offset_dims=(), collapsed_slice_dims=(0,), start_index_map=(0,))
    idx = jnp.full((v.shape[0],), i, jnp.int32)
    return lax.gather(v, idx[:, None], dn, (1,),
                      mode=lax.GatherScatterMode.PROMISE_IN_BOUNDS)


@functools.partial(jax.jit, static_argnames=("n", "e"))
def _sc_compact(src, dst, ew, n, e):
    info = plsc.get_sparse_core_info()
    nc, ns, nl = info.num_cores, info.num_subcores, info.num_lanes
    half = n // nc
    quarter = half // _NQ
    eps = e // ns
    n_seg = eps // _SEG
    n_grp = _SEG // nl
    rnd = _BK * _CH
    cap = eps + 2 * rnd

    mesh = plsc.VectorSubcoreMesh(core_axis_name="c", subcore_axis_name="s")

    @functools.partial(
        pl.kernel,
        mesh=mesh,
        compiler_params=pltpu.CompilerParams(needs_layout_passes=False),
        out_type=(
            jax.ShapeDtypeStruct((nc * ns * _NQ * cap,), jnp.int32),
            jax.ShapeDtypeStruct((nc * ns * _NQ * cap,), jnp.float32),
            jax.ShapeDtypeStruct((nc * ns * _NQ * nl,), jnp.int32),
        ),
        scratch_types=[
            pltpu.VMEM((_SEG,), jnp.int32),
            pltpu.VMEM((_SEG,), jnp.int32),
            pltpu.VMEM((_SEG,), jnp.float32),
            [pltpu.VMEM((cap,), jnp.int32) for _ in range(_NQ)],
            [pltpu.VMEM((cap,), jnp.float32) for _ in range(_NQ)],
            pltpu.VMEM((nl,), jnp.int32),
        ],
    )
    def k(src_hbm, dst_hbm, ew_hbm, out_p, out_w, out_n,
          seg_s, seg_d, seg_w, pk_c, ew_c, cnt_v):
        cid = lax.axis_index("c")
        sid = lax.axis_index("s")
        lo = cid * half

        def _seg(s, cursors):
            base = sid * eps + s * _SEG
            pltpu.sync_copy(src_hbm.at[pl.ds(base, _SEG)], seg_s)
            pltpu.sync_copy(dst_hbm.at[pl.ds(base, _SEG)], seg_d)
            pltpu.sync_copy(ew_hbm.at[pl.ds(base, _SEG)], seg_w)

            def _grp(g, curs):
                dv = seg_d[pl.ds(g * nl, nl)] - lo
                sv = seg_s[pl.ds(g * nl, nl)]
                wv = seg_w[pl.ds(g * nl, nl)]
                new = []
                for q in range(_NQ):
                    dq = dv - q * quarter
                    m = (dq >= 0) & (dq < quarter)
                    pv = sv | (dq << _SB)
                    plsc.store_compressed(pk_c[q].at[pl.ds(curs[q], nl)],
                                          pv, mask=m)
                    plsc.store_compressed(ew_c[q].at[pl.ds(curs[q], nl)],
                                          wv, mask=m)
                    new.append(
                        curs[q]
                        + jnp.max(plsc.all_reduce_population_count(m)))
                return tuple(new)

            return lax.fori_loop(0, n_grp, _grp, cursors)

        cursors = lax.fori_loop(0, n_seg, _seg,
                                tuple(jnp.int32(0) for _ in range(_NQ)))

        pad = jnp.full((nl,), quarter << _SB, jnp.int32)
        zero = jnp.zeros((nl,), jnp.float32)
        for q in range(_NQ):
            for b in range(rnd // nl):
                pk_c[q][pl.ds(cursors[q] + b * nl, nl)] = pad
                ew_c[q][pl.ds(cursors[q] + b * nl, nl)] = zero
            n_chunks = _BK * ((cursors[q] + rnd) // rnd)
            cnt_v[pl.ds(0, nl)] = jnp.full((nl,), n_chunks, jnp.int32)
            slot = (cid * ns + sid) * _NQ + q
            pltpu.sync_copy(pk_c[q], out_p.at[pl.ds(slot * cap, cap)])
            pltpu.sync_copy(ew_c[q], out_w.at[pl.ds(slot * cap, cap)])
            pltpu.sync_copy(cnt_v, out_n.at[pl.ds(slot * nl, nl)])

    return k(src, dst, ew)


@functools.partial(jax.jit, static_argnames=("n", "d"))
def _sc_aggregate(feat, pk_in, ew_in, cnt_in, n, d):
    info = plsc.get_sparse_core_info()
    nc, ns, nl = info.num_cores, info.num_subcores, info.num_lanes
    half = n // nc
    quarter = half // _NQ
    acc_rows = ((quarter + 1) + ns * 8 - 1) // (ns * 8) * (ns * 8)
    srps = acc_rows // ns
    cap = pk_in.shape[0] // (nc * ns * _NQ)

    mesh = plsc.VectorSubcoreMesh(core_axis_name="c", subcore_axis_name="s")

    @functools.partial(
        pl.kernel,
        mesh=mesh,
        compiler_params=pltpu.CompilerParams(needs_layout_passes=False),
        out_type=jax.ShapeDtypeStruct((nc * _NQ * ns * srps, d), jnp.float32),
        scratch_types=[
            pltpu.VMEM((2 * _BK * _CH,), jnp.int32),
            pltpu.VMEM((2 * _BK * _CH,), jnp.float32),
            pltpu.VMEM((_NG, _CH), jnp.int32),
            pltpu.VMEM((_NG, _CH), jnp.int32),
            pltpu.VMEM((_NG, _CH, d), jnp.float32),
            pltpu.VMEM((nl,), jnp.int32),
            pltpu.VMEM_SHARED((n, d), jnp.float32),
            pltpu.VMEM_SHARED((acc_rows, d), jnp.float32),
            pltpu.SemaphoreType.DMA((2,)),
            pltpu.SemaphoreType.DMA((_NG,)),
            pltpu.SemaphoreType.DMA((_NG,)),
        ],
    )
    def k(feat_hbm, pk_hbm, ew_hbm, cnt_hbm, out_hbm,
          pk_k, ew_k, src_k, dst_k, grows, cnt_v,
          feat_s, agg_s, sem_p, sem_g, sem_s):
        cid = lax.axis_index("c")
        sid = lax.axis_index("s")

        fbig = (n // ns + 15) // 16 * 16
        flast = n - (ns - 1) * fbig

        @pl.when(sid < ns - 1)
        def _():
            pltpu.sync_copy(feat_hbm.at[pl.ds(sid * fbig, fbig)],
                            feat_s.at[pl.ds(sid * fbig, fbig)])

        @pl.when(sid == ns - 1)
        def _():
            pltpu.sync_copy(feat_hbm.at[pl.ds((ns - 1) * fbig, flast)],
                            feat_s.at[pl.ds((ns - 1) * fbig, flast)])

        zero = jnp.zeros((nl,), jnp.float32)

        def _zrow(i, carry):
            for j in range(d // nl):
                grows[0, i, pl.ds(j * nl, nl)] = zero
            return carry

        lax.fori_loop(0, _CH, _zrow, 0)

        def _pass(q, carry):
            slot = (cid * ns + sid) * _NQ + q
            ebase = slot * cap

            pltpu.sync_copy(cnt_hbm.at[pl.ds(slot * nl, nl)], cnt_v)
            n_chunks = jnp.max(cnt_v[pl.ds(0, nl)])

            def _zstripe(kk, c2):
                pltpu.sync_copy(
                    grows.at[0],
                    agg_s.at[pl.ds(sid * srps + kk * _CH, _CH)])
                return c2

            lax.fori_loop(0, srps // _CH, _zstripe, 0)
            zrem = srps - (srps // _CH) * _CH
            if zrem:
                pltpu.sync_copy(
                    grows.at[0].at[pl.ds(0, zrem)],
                    agg_s.at[pl.ds(sid * srps + (srps // _CH) * _CH, zrem)])
            plsc.subcore_barrier()

            def _stage_blk(kb):
                sl = lax.rem(kb, 2)
                base = ebase + kb * _BK * _CH
                pltpu.async_copy(ew_hbm.at[pl.ds(base, _BK * _CH)],
                                 ew_k.at[pl.ds(sl * _BK * _CH, _BK * _CH)],
                                 sem_p.at[sl])
                pltpu.async_copy(pk_hbm.at[pl.ds(base, _BK * _CH)],
                                 pk_k.at[pl.ds(sl * _BK * _CH, _BK * _CH)],
                                 sem_p.at[sl])

            def _wait_blk(kb):
                sl = lax.rem(kb, 2)
                base = ebase + kb * _BK * _CH
                pltpu.make_async_copy(
                    ew_hbm.at[pl.ds(base, _BK * _CH)],
                    ew_k.at[pl.ds(sl * _BK * _CH, _BK * _CH)],
                    sem_p.at[sl]).wait()
                pltpu.make_async_copy(
                    pk_hbm.at[pl.ds(base, _BK * _CH)],
                    pk_k.at[pl.ds(sl * _BK * _CH, _BK * _CH)],
                    sem_p.at[sl]).wait()

            def _fire_gather(t):
                b = lax.rem(t, _NG)
                sl = lax.rem(t // _BK, 2)
                off = lax.rem(t, _BK) * _CH

                def _unp(kk, c2):
                    pv = pk_k[pl.ds(sl * _BK * _CH + off + kk * nl, nl)]
                    src_k[b, pl.ds(kk * nl, nl)] = pv & ((1 << _SB) - 1)
                    dst_k[b, pl.ds(kk * nl, nl)] = pv >> _SB
                    return c2

                lax.fori_loop(0, _CH // nl, _unp, 0)
                pltpu.async_copy(feat_s.at[src_k.at[b]], grows.at[b],
                                 sem_g.at[b])

            _stage_blk(0)

            @pl.when(n_chunks > _BK)
            def _():
                _stage_blk(1)

            _wait_blk(0)
            _fire_gather(0)

            def _chunk(t, c2):
                b = lax.rem(t, _NG)
                o = lax.rem(t + 1, _NG)

                @pl.when((lax.rem(t, _BK) == 0) & (t > 0)
                         & ((t // _BK + 1) * _BK < n_chunks))
                def _():
                    _stage_blk(t // _BK + 1)

                @pl.when(t + 1 >= _NG)
                def _():
                    pltpu.make_async_copy(grows.at[o],
                                          agg_s.at[dst_k.at[o]],
                                          sem_s.at[o]).wait()

                @pl.when(t + 1 < n_chunks)
                def _():
                    @pl.when(lax.rem(t + 1, _BK) == 0)
                    def _():
                        _wait_blk((t + 1) // _BK)

                    _fire_gather(t + 1)

                pltpu.make_async_copy(feat_s.at[src_k.at[b]], grows.at[b],
                                      sem_g.at[b]).wait()

                sl = lax.rem(t // _BK, 2)
                woff = lax.rem(t, _BK) * _CH

                def _grp2(g, c3):
                    wv = ew_k[pl.ds(sl * _BK * _CH + woff + g * nl, nl)]
                    for il in range(nl):
                        w = _lane_bcast(wv, il)
                        r = g * nl + il
                        for j in range(d // nl):
                            grows[b, r, pl.ds(j * nl, nl)] = (
                                grows[b, r, pl.ds(j * nl, nl)] * w)
                    return c3

                lax.fori_loop(0, _CH // nl, _grp2, 0)
                pltpu.async_copy(grows.at[b], agg_s.at[dst_k.at[b]],
                                 sem_s.at[b], add=True)
                return c2

            lax.fori_loop(0, n_chunks, _chunk, 0)

            def _dr(i, c2):
                tt = n_chunks - 1 - i
                bb = lax.rem(tt, _NG)
                pltpu.make_async_copy(grows.at[bb], agg_s.at[dst_k.at[bb]],
                                      sem_s.at[bb]).wait()
                return c2

            lax.fori_loop(0, _NG - 1, _dr, 0)
            plsc.subcore_barrier()

            obase = ((cid * _NQ + q) * ns + sid) * srps
            pltpu.sync_copy(agg_s.at[pl.ds(sid * srps, srps)],
                            out_hbm.at[pl.ds(obase, srps)])
            plsc.subcore_barrier()

            @pl.when(q + 1 < _NQ)
            def _():
                lax.fori_loop(0, _CH, _zrow, 0)

            return carry

        lax.fori_loop(0, _NQ, _pass, 0)

    return k(feat, pk_in, ew_in, cnt_in)


def _tc_finish(features, agg, kern, bias, skip):
    n, d = features.shape
    br = 2000

    def body(f_ref, a_ref, k_ref, b_ref, s_ref, o_ref):
        k = k_ref[...]
        x = (jnp.dot(f_ref[...], k, preferred_element_type=jnp.float32)
             * s_ref[...]
             + jnp.dot(a_ref[...], k, preferred_element_type=jnp.float32)
             + b_ref[...])
        o_ref[...] = _SELU_SCALE * jnp.where(
            x > 0, x, _SELU_ALPHA * (jnp.exp(x) - 1.0))

    return pl.pallas_call(
        body,
        grid=(n // br,),
        in_specs=[
            pl.BlockSpec((br, d), lambda i: (i, 0)),
            pl.BlockSpec((br, d), lambda i: (i, 0)),
            pl.BlockSpec((d, d), lambda i: (0, 0)),
            pl.BlockSpec((1, d), lambda i: (0, 0)),
            pl.BlockSpec((1, d), lambda i: (0, 0)),
        ],
        out_specs=pl.BlockSpec((br, d), lambda i: (i, 0)),
        out_shape=jax.ShapeDtypeStruct((n, d), jnp.float32),
    )(features, agg, kern, bias.reshape(1, d), skip.reshape(1, d))


def kernel(features, edge_index, edge_weight, kernel, bias, skip_weight):
    n, d = features.shape
    e = edge_index.shape[1]
    pk, wts, cnt = _sc_compact(edge_index[1], edge_index[0], edge_weight,
                               n, e)
    parts = _sc_aggregate(features, pk, wts, cnt, n, d)
    nq = 2 * _NQ
    quarter = n // nq
    parts = parts.reshape(nq, -1, d)
    agg = jnp.concatenate([parts[i, :quarter] for i in range(nq)], axis=0)
    return _tc_finish(features, agg, kernel, bias, skip_weight)


# speedup vs baseline: 1.6150x; 1.6150x over previous
import functools

import jax
import jax.numpy as jnp
from jax import lax
from jax.experimental import pallas as pl
from jax.experimental.pallas import tpu as pltpu
from jax.experimental.pallas import tpu_sc as plsc

_SELU_ALPHA = 1.6732632423543772
_SELU_SCALE = 1.0507009873554805

_CH = 80
_SEG = 2000


def _lane_bcast(v, i):
    dn = lax.GatherDimensionNumbers(
        offset_dims=(), collapsed_slice_dims=(0,), start_index_map=(0,))
    idx = jnp.full((v.shape[0],), i, jnp.int32)
    return lax.gather(v, idx[:, None], dn, (1,),
                      mode=lax.GatherScatterMode.PROMISE_IN_BOUNDS)


@functools.partial(jax.jit, static_argnames=("n", "d", "e"))
def _sc_aggregate(features, src, dst, ew, n, d, e):
    info = plsc.get_sparse_core_info()
    nc, ns, nl = info.num_cores, info.num_subcores, info.num_lanes
    half = n // nc
    acc_rows = ((half + _CH) + ns * 8 - 1) // (ns * 8) * (ns * 8)
    srps = acc_rows // ns
    eps = e // ns
    n_seg = eps // _SEG
    n_grp = _SEG // nl
    cmax = eps + 4 * _CH

    mesh = plsc.VectorSubcoreMesh(core_axis_name="c", subcore_axis_name="s")

    @functools.partial(
        pl.kernel,
        mesh=mesh,
        compiler_params=pltpu.CompilerParams(needs_layout_passes=False),
        out_type=jax.ShapeDtypeStruct((nc, ns, srps, d), jnp.float32),
        scratch_types=[
            pltpu.VMEM((_SEG,), jnp.int32),
            pltpu.VMEM((_SEG,), jnp.int32),
            pltpu.VMEM((_SEG,), jnp.float32),
            pltpu.VMEM((cmax,), jnp.int32),
            pltpu.VMEM((cmax,), jnp.int32),
            pltpu.VMEM((cmax,), jnp.float32),
            pltpu.VMEM((2, _CH), jnp.int32),
            pltpu.VMEM((2, _CH), jnp.int32),
            pltpu.VMEM((_CH, d), jnp.float32),
            pltpu.VMEM((_CH, d), jnp.float32),
            pltpu.VMEM_SHARED((acc_rows, d), jnp.float32),
            pltpu.SemaphoreType.DMA,
            pltpu.SemaphoreType.DMA,
            pltpu.SemaphoreType.DMA,
            pltpu.SemaphoreType.DMA,
        ],
    )
    def sc_kernel(feat_hbm, src_hbm, dst_hbm, ew_hbm, out_hbm,
                  seg_s, seg_d, seg_w, src_c, dst_c, ew_c,
                  src_k, dst_k, rows0, rows1, agg_s,
                  sem_g0, sem_g1, sem_s0, sem_s1):
        cid = lax.axis_index("c")
        sid = lax.axis_index("s")
        lo = cid * half
        rows = (rows0, rows1)
        sem_g = (sem_g0, sem_g1)
        sem_s = (sem_s0, sem_s1)

        zero = jnp.zeros((nl,), jnp.float32)

        def _zrow(i, carry):
            for j in range(d // nl):
                rows0[i, pl.ds(j * nl, nl)] = zero
            return carry

        lax.fori_loop(0, _CH, _zrow, 0)
        done = 0
        while done < srps:
            step = min(_CH, srps - done)
            pltpu.sync_copy(rows0.at[pl.ds(0, step)],
                            agg_s.at[pl.ds(sid * srps + done, step)])
            done += step

        def _seg(s, cursor):
            base = sid * eps + s * _SEG
            pltpu.sync_copy(src_hbm.at[pl.ds(base, _SEG)], seg_s)
            pltpu.sync_copy(dst_hbm.at[pl.ds(base, _SEG)], seg_d)
            pltpu.sync_copy(ew_hbm.at[pl.ds(base, _SEG)], seg_w)

            def _grp(g, cur):
                dv = seg_d[pl.ds(g * nl, nl)] - lo
                sv = seg_s[pl.ds(g * nl, nl)]
                wv = seg_w[pl.ds(g * nl, nl)]
                m = (dv >= 0) & (dv < half)
                plsc.store_compressed(dst_c.at[pl.ds(cur, nl)], dv, mask=m)
                plsc.store_compressed(src_c.at[pl.ds(cur, nl)], sv, mask=m)
                plsc.store_compressed(ew_c.at[pl.ds(cur, nl)], wv, mask=m)
                return cur + jnp.max(plsc.all_reduce_population_count(m))

            return lax.fori_loop(0, n_grp, _grp, cursor)

        cursor = lax.fori_loop(0, n_seg, _seg, jnp.int32(0))

        for b in range(2 * _CH // nl):
            dst_c[pl.ds(cursor + b * nl, nl)] = jnp.full((nl,), half, jnp.int32)
            src_c[pl.ds(cursor + b * nl, nl)] = jnp.zeros((nl,), jnp.int32)
            ew_c[pl.ds(cursor + b * nl, nl)] = zero
        n_pairs = (cursor + 2 * _CH - 1) // (2 * _CH)
        n_chunks = 2 * n_pairs
        plsc.subcore_barrier()

        def _stage_idx(t, b):
            for k in range(_CH // nl):
                src_k[b, pl.ds(k * nl, nl)] = src_c[pl.ds(t * _CH + k * nl, nl)]
                dst_k[b, pl.ds(k * nl, nl)] = dst_c[pl.ds(t * _CH + k * nl, nl)]

        def _issue_gather(b):
            return pltpu.async_copy(feat_hbm.at[src_k.at[b]], rows[b],
                                    sem_g[b])

        def _scale(t, b):
            rb = rows[b]

            def _grp2(g, c2):
                wv = ew_c[pl.ds(t * _CH + g * nl, nl)]
                for il in range(nl):
                    w = _lane_bcast(wv, il)
                    r = g * nl + il
                    for j in range(d // nl):
                        rb[r, pl.ds(j * nl, nl)] = rb[r, pl.ds(j * nl, nl)] * w
                return c2

            lax.fori_loop(0, _CH // nl, _grp2, 0)

        _stage_idx(0, 0)
        _issue_gather(0)

        def _pair(p, carry):
            for b in range(2):
                t = 2 * p + b
                o = 1 - b
                pltpu.make_async_copy(feat_hbm.at[src_k.at[b]], rows[b],
                                      sem_g[b]).wait()
                @pl.when(t > 0)
                def _():
                    pltpu.make_async_copy(rows[o], agg_s.at[dst_k.at[o]],
                                          sem_s[o]).wait()

                @pl.when(t + 1 < n_chunks)
                def _():
                    _stage_idx(t + 1, o)
                    _issue_gather(o)

                _scale(t, b)
                pltpu.async_copy(rows[b], agg_s.at[dst_k.at[b]], sem_s[b],
                                 add=True)
            return carry

        lax.fori_loop(0, n_pairs, _pair, 0)
        pltpu.make_async_copy(rows[1], agg_s.at[dst_k.at[1]], sem_s[1]).wait()
        plsc.subcore_barrier()

        pltpu.sync_copy(agg_s.at[pl.ds(sid * srps, srps)], out_hbm.at[cid, sid])

    return sc_kernel(features, src, dst, ew)


def _tc_finish(features, agg, kern, bias, skip):
    n, d = features.shape
    br = 2000

    def body(f_ref, a_ref, k_ref, b_ref, s_ref, o_ref):
        k = k_ref[...]
        x = (jnp.dot(f_ref[...], k, preferred_element_type=jnp.float32)
             * s_ref[...]
             + jnp.dot(a_ref[...], k, preferred_element_type=jnp.float32)
             + b_ref[...])
        o_ref[...] = _SELU_SCALE * jnp.where(
            x > 0, x, _SELU_ALPHA * (jnp.exp(x) - 1.0))

    return pl.pallas_call(
        body,
        grid=(n // br,),
        in_specs=[
            pl.BlockSpec((br, d), lambda i: (i, 0)),
            pl.BlockSpec((br, d), lambda i: (i, 0)),
            pl.BlockSpec((d, d), lambda i: (0, 0)),
            pl.BlockSpec((1, d), lambda i: (0, 0)),
            pl.BlockSpec((1, d), lambda i: (0, 0)),
        ],
        out_specs=pl.BlockSpec((br, d), lambda i: (i, 0)),
        out_shape=jax.ShapeDtypeStruct((n, d), jnp.float32),
    )(features, agg, kern, bias.reshape(1, d), skip.reshape(1, d))


def kernel(features, edge_index, edge_weight, kernel, bias, skip_weight):
    n, d = features.shape
    e = edge_index.shape[1]
    parts = _sc_aggregate(features, edge_index[1], edge_index[0],
                          edge_weight, n, d, e)
    nc = parts.shape[0]
    half = n // nc
    parts = parts.reshape(nc, -1, d)
    agg = jnp.concatenate([parts[c, :half] for c in range(nc)], axis=0)
    return _tc_finish(features, agg, kernel, bias, skip_weight)
